# baseline (speedup 1.0000x reference)
import functools

import jax
import jax.numpy as jnp
from jax import lax
from jax.experimental import pallas as pl
from jax.experimental.pallas import tpu as pltpu
from jax.experimental.pallas import tpu_sc as plsc

N = 1_000_000
K = 950_000
NW = 32
C = 31_264
NPAD = NW * C
NB1 = 2048
NB2 = 1024
NB3 = 1024
SR = 2048
CAP = SR - 16
HCHUNK = C // 2

_mesh = plsc.VectorSubcoreMesh(core_axis_name="c", subcore_axis_name="s")
_params = pltpu.CompilerParams(needs_layout_passes=False)


def _wid():
    return lax.axis_index("s") * 2 + lax.axis_index("c")


def _lane():
    return lax.broadcasted_iota(jnp.int32, (16,), 0)


def _zero_ref(ref, nwords):
    z = jnp.zeros((16,), jnp.int32)

    @plsc.parallel_loop(0, nwords // 16, unroll=8)
    def _(i):
        ref[pl.ds(i * 16, 16)] = z


def _scan_rows(hin, nrows, nbins, kth):
    zeros = jnp.zeros((16,), jnp.int32)
    ones = jnp.ones((16,), jnp.int32)

    @plsc.parallel_loop(
        0, nbins // 16, carry=(jnp.int32(0), jnp.int32(0), jnp.int32(0)))
    def body(g, carry):
        total, b_acc, pre_acc = carry
        acc = zeros
        for r in range(nrows):
            acc = acc + hin[pl.ds(r * nbins + g * 16, 16)]
        cs = plsc.cumsum(acc)
        lt = (total + cs) < kth
        b_acc = b_acc + jnp.sum(jnp.where(lt, ones, zeros))
        pre_acc = pre_acc + jnp.sum(jnp.where(lt, acc, zeros))
        total = total + jnp.sum(acc)
        return total, b_acc, pre_acc

    _, b, pre = body
    return b, pre


def _reduce_banked(hb, hrow, nbins):
    zeros = jnp.zeros((16,), jnp.int32)

    @plsc.parallel_loop(0, nbins // 16, unroll=2)
    def _(g):
        acc = zeros
        for l in range(16):
            acc = acc + hb[pl.ds(l * nbins + g * 16, 16)]
        hrow[pl.ds(g * 16, 16)] = acc


@functools.partial(
    pl.kernel,
    mesh=_mesh,
    compiler_params=_params,
    out_type=(
        jax.ShapeDtypeStruct((NPAD,), jnp.float32),
        jax.ShapeDtypeStruct((NW * NB1,), jnp.int32),
    ),
    scratch_types=[
        pltpu.VMEM((C,), jnp.float32),
        pltpu.VMEM((C,), jnp.float32),
        pltpu.VMEM((16 * NB1,), jnp.int32),
        pltpu.VMEM((NB1,), jnp.int32),
        pltpu.SemaphoreType.DMA,
        pltpu.SemaphoreType.DMA,
    ],
)
def _k1(p_hbm, t_hbm, err_hbm, h1_hbm, pv, tv, hb, hrow, sem1, sem2):
    wid = _wid()
    base = wid * C
    cp1 = pltpu.async_copy(p_hbm.at[pl.ds(base, C)], pv, sem1)
    cp2 = pltpu.async_copy(t_hbm.at[pl.ds(base, C)], tv, sem2)
    _zero_ref(hb, 16 * NB1)
    cp1.wait()
    cp2.wait()
    lane_off = _lane() * NB1
    ones = jnp.ones((16,), jnp.int32)

    @plsc.parallel_loop(0, C // 16, unroll=8)
    def _(i):
        p = pv[pl.ds(i * 16, 16)]
        t = tv[pl.ds(i * 16, 16)]
        e = jnp.abs(p - t)
        pv[pl.ds(i * 16, 16)] = e
        u = lax.bitcast_convert_type(e, jnp.int32)
        off = lane_off + (u >> 20)
        plsc.addupdate_scatter(hb, [off], ones)

    _reduce_banked(hb, hrow, NB1)
    pltpu.sync_copy(pv, err_hbm.at[pl.ds(base, C)])
    pltpu.sync_copy(hrow, h1_hbm.at[pl.ds(wid * NB1, NB1)])


@functools.partial(
    pl.kernel,
    mesh=_mesh,
    compiler_params=_params,
    out_type=(
        jax.ShapeDtypeStruct((NW * SR,), jnp.int32),
        jax.ShapeDtypeStruct((NW * 16,), jnp.int32),
        jax.ShapeDtypeStruct((16,), jnp.int32),
    ),
    scratch_types=[
        pltpu.VMEM((C,), jnp.float32),
        pltpu.VMEM((NW * NB1,), jnp.int32),
        pltpu.VMEM((SR,), jnp.int32),
        pltpu.VMEM((16,), jnp.int32),
        pltpu.SemaphoreType.DMA,
        pltpu.SemaphoreType.DMA,
    ],
)
def _k2(err_hbm, h1_hbm, surv_hbm, cnt_hbm, st_hbm, ev, hin, sv, stv,
        sem1, sem2):
    wid = _wid()
    base = wid * C
    cp1 = pltpu.async_copy(err_hbm.at[pl.ds(base, C)], ev, sem1)
    cp2 = pltpu.async_copy(h1_hbm, hin, sem2)
    cp2.wait()
    b0, pre0 = _scan_rows(hin, NW, NB1, K)
    kp = K - pre0
    cp1.wait()
    lane = _lane()
    zeros = jnp.zeros((16,), jnp.int32)
    ones = jnp.ones((16,), jnp.int32)
    capv = zeros + CAP

    @plsc.parallel_loop(0, C // 16, carry=zeros)
    def offv(i, offv):
        e = ev[pl.ds(i * 16, 16)]
        u = lax.bitcast_convert_type(e, jnp.int32)
        m = (u >> 20) == b0
        mi = jnp.where(m, ones, zeros)
        pos = offv + plsc.cumsum(mi) - mi
        plsc.store_scatter(sv, [jnp.minimum(pos, capv)], u, mask=m)
        return offv + plsc.all_reduce_population_count(m)

    pltpu.sync_copy(sv, surv_hbm.at[pl.ds(wid * SR, SR)])
    stv[...] = offv
    pltpu.sync_copy(stv, cnt_hbm.at[pl.ds(wid * 16, 16)])

    @pl.when(wid == 0)
    def _():
        stv[...] = jnp.where(lane == 0, b0, jnp.where(lane == 1, kp, zeros))
        pltpu.sync_copy(stv, st_hbm)


@functools.partial(
    pl.kernel,
    mesh=_mesh,
    compiler_params=_params,
    out_type=jax.ShapeDtypeStruct((16,), jnp.float32),
    scratch_types=[
        pltpu.VMEM((NW * SR,), jnp.int32),
        pltpu.VMEM((HCHUNK,), jnp.float32),
        pltpu.VMEM((16 * NB2,), jnp.int32),
        pltpu.VMEM((NB2,), jnp.int32),
        pltpu.VMEM((NW * 16,), jnp.int32),
        pltpu.VMEM((16,), jnp.int32),
        pltpu.VMEM((16,), jnp.float32),
        pltpu.SemaphoreType.DMA,
    ],
)
def _kc(err_hbm, surv_hbm, cnt_hbm, st_hbm, res_hbm,
        sb, fb, hb, hrow, cntv, stv, outv, sem):
    wid = _wid()

    @pl.when(wid == 0)
    def _():
        pltpu.sync_copy(cnt_hbm, cntv)
        pltpu.sync_copy(st_hbm, stv)
        svec = stv[...]
        b0 = svec[0]
        kp = svec[1]
        lane = _lane()
        zeros = jnp.zeros((16,), jnp.int32)
        ones = jnp.ones((16,), jnp.int32)

        copies = [
            pltpu.async_copy(
                surv_hbm.at[pl.ds(r * SR, SR)], sb.at[pl.ds(r * SR, SR)], sem)
            for r in range(NW)
        ]
        mx = zeros
        for r in range(NW):
            mx = jnp.maximum(mx, cntv[pl.ds(r * 16, 16)])
        ov = jnp.any(mx > CAP)
        for cp in copies:
            cp.wait()

        def survivors_hist(match_fn):
            for r in range(NW):
                cnt = cntv[pl.ds(r * 16, 16)][0]

                @functools.partial(lax.fori_loop, 0, (cnt + 15) >> 4, init_val=0)
                def _(j, _ignored):
                    u = sb[pl.ds(r * SR + j * 16, 16)]
                    valid = (j * 16 + lane) < cnt
                    m, bkt = match_fn(u)
                    plsc.addupdate_scatter(
                        hb, [lane * NB2 + bkt], ones, mask=m & valid)
                    return 0

        def full_rescan_hist(match_fn):
            def rbody(r, _):
                for c in range(2):
                    pltpu.sync_copy(
                        err_hbm.at[pl.ds(r * C + c * HCHUNK, HCHUNK)], fb)

                    @functools.partial(
                        lax.fori_loop, 0, HCHUNK // 16, init_val=0)
                    def _(j, _ignored):
                        e = fb[pl.ds(j * 16, 16)]
                        u = lax.bitcast_convert_type(e, jnp.int32)
                        m, bkt = match_fn(u)
                        plsc.addupdate_scatter(
                            hb, [lane * NB2 + bkt], ones, mask=m)
                        return 0
                return 0

            lax.fori_loop(0, NW, rbody, 0)

        def select_pass(match_fn, kth):
            _zero_ref(hb, 16 * NB2)
            pl.when(~ov)(lambda: survivors_hist(match_fn))
            pl.when(ov)(lambda: full_rescan_hist(match_fn))
            _reduce_banked(hb, hrow, NB2)
            return _scan_rows(hrow, 1, NB2, kth)

        b1, pre1 = select_pass(
            lambda u: (((u >> 20) == b0), (u >> 10) & (NB2 - 1)), kp)
        p01 = b0 * NB2 + b1
        kpp = kp - pre1
        b2, _unused = select_pass(
            lambda u: (((u >> 10) == p01), u & (NB3 - 1)), kpp)
        bits = p01 * NB3 + b2
        outv[...] = lax.bitcast_convert_type(zeros + bits, jnp.float32)
        pltpu.sync_copy(outv, res_hbm)


def kernel(pred, target):
    padp = jnp.concatenate(
        [pred, jnp.full((NPAD - N,), jnp.inf, jnp.float32)])
    padt = jnp.concatenate([target, jnp.zeros((NPAD - N,), jnp.float32)])
    err, h1 = _k1(padp, padt)
    surv, cnt, st = _k2(err, h1)
    res = _kc(err, surv, cnt, st)
    return res[0]


# speedup vs baseline: 1.1040x; 1.1040x over previous
import functools

import jax
import jax.numpy as jnp
from jax import lax
from jax.experimental import pallas as pl
from jax.experimental.pallas import tpu as pltpu
from jax.experimental.pallas import tpu_sc as plsc

N = 1_000_000
K = 950_000
NW = 32
C = 31_264
NPAD = NW * C
NB1 = 2048
NB2 = 1024
NB3 = 1024
SR = 2048
CAP = SR - 16
HCHUNK = C // 2

_mesh = plsc.VectorSubcoreMesh(core_axis_name="c", subcore_axis_name="s")
_params = pltpu.CompilerParams(needs_layout_passes=False)


def _wid():
    return lax.axis_index("s") * 2 + lax.axis_index("c")


def _lane():
    return lax.broadcasted_iota(jnp.int32, (16,), 0)


def _zero_ref(ref, nwords):
    z = jnp.zeros((16,), jnp.int32)

    @plsc.parallel_loop(0, nwords // 16, unroll=8)
    def _(i):
        ref[pl.ds(i * 16, 16)] = z


def _scan_rows(hin, nrows, nbins, kth):
    zeros = jnp.zeros((16,), jnp.int32)
    ones = jnp.ones((16,), jnp.int32)

    @plsc.parallel_loop(
        0, nbins // 16, carry=(jnp.int32(0), jnp.int32(0), jnp.int32(0)))
    def body(g, carry):
        total, b_acc, pre_acc = carry
        acc = zeros
        for r in range(nrows):
            acc = acc + hin[pl.ds(r * nbins + g * 16, 16)]
        cs = plsc.cumsum(acc)
        lt = (total + cs) < kth
        b_acc = b_acc + jnp.sum(jnp.where(lt, ones, zeros))
        pre_acc = pre_acc + jnp.sum(jnp.where(lt, acc, zeros))
        total = total + jnp.sum(acc)
        return total, b_acc, pre_acc

    _, b, pre = body
    return b, pre


def _reduce_banked(hb, hrow, nbins):
    zeros = jnp.zeros((16,), jnp.int32)

    @plsc.parallel_loop(0, nbins // 16, unroll=2)
    def _(g):
        acc = zeros
        for l in range(16):
            acc = acc + hb[pl.ds(l * nbins + g * 16, 16)]
        hrow[pl.ds(g * 16, 16)] = acc


@functools.partial(
    pl.kernel,
    mesh=_mesh,
    compiler_params=_params,
    out_type=(
        jax.ShapeDtypeStruct((NPAD,), jnp.float32),
        jax.ShapeDtypeStruct((NW * NB1,), jnp.int32),
    ),
    scratch_types=[
        pltpu.VMEM((C,), jnp.float32),
        pltpu.VMEM((C,), jnp.float32),
        pltpu.VMEM((16 * NB1,), jnp.int32),
        pltpu.VMEM((NB1,), jnp.int32),
        pltpu.SemaphoreType.DMA,
        pltpu.SemaphoreType.DMA,
    ],
)
def _k1(p_hbm, t_hbm, err_hbm, h1_hbm, pv, tv, hb, hrow, sem1, sem2):
    wid = _wid()
    base = wid * C
    cp1 = pltpu.async_copy(p_hbm.at[pl.ds(base, C)], pv, sem1)
    cp2 = pltpu.async_copy(t_hbm.at[pl.ds(base, C)], tv, sem2)
    _zero_ref(hb, 16 * NB1)
    cp1.wait()
    cp2.wait()
    lane_off = _lane() * NB1
    ones = jnp.ones((16,), jnp.int32)

    @plsc.parallel_loop(0, C // 16, unroll=8)
    def _(i):
        p = pv[pl.ds(i * 16, 16)]
        t = tv[pl.ds(i * 16, 16)]
        e = jnp.abs(p - t)
        pv[pl.ds(i * 16, 16)] = e
        u = lax.bitcast_convert_type(e, jnp.int32)
        off = lane_off + (u >> 20)
        plsc.addupdate_scatter(hb, [off], ones)

    _reduce_banked(hb, hrow, NB1)
    pltpu.sync_copy(pv, err_hbm.at[pl.ds(base, C)])
    pltpu.sync_copy(hrow, h1_hbm.at[pl.ds(wid * NB1, NB1)])


@functools.partial(
    pl.kernel,
    mesh=_mesh,
    compiler_params=_params,
    out_type=(
        jax.ShapeDtypeStruct((NW * SR,), jnp.int32),
        jax.ShapeDtypeStruct((NW * 16,), jnp.int32),
        jax.ShapeDtypeStruct((16,), jnp.int32),
    ),
    scratch_types=[
        pltpu.VMEM((C,), jnp.float32),
        pltpu.VMEM((NW * NB1,), jnp.int32),
        pltpu.VMEM((SR,), jnp.int32),
        pltpu.VMEM((16,), jnp.int32),
        pltpu.SemaphoreType.DMA,
        pltpu.SemaphoreType.DMA,
    ],
)
def _k2(err_hbm, h1_hbm, surv_hbm, cnt_hbm, st_hbm, ev, hin, sv, stv,
        sem1, sem2):
    wid = _wid()
    base = wid * C
    cp1 = pltpu.async_copy(err_hbm.at[pl.ds(base, C)], ev, sem1)
    cp2 = pltpu.async_copy(h1_hbm, hin, sem2)
    cp2.wait()
    b0, pre0 = _scan_rows(hin, NW, NB1, K)
    kp = K - pre0
    cp1.wait()
    lane = _lane()
    zeros = jnp.zeros((16,), jnp.int32)
    ones = jnp.ones((16,), jnp.int32)
    capv = zeros + CAP

    @plsc.parallel_loop(0, C // 16, carry=zeros)
    def offv(i, offv):
        e = ev[pl.ds(i * 16, 16)]
        u = lax.bitcast_convert_type(e, jnp.int32)
        m = (u >> 20) == b0
        mi = jnp.where(m, ones, zeros)
        pos = offv + plsc.cumsum(mi) - mi
        plsc.store_scatter(sv, [jnp.minimum(pos, capv)], u, mask=m)
        return offv + plsc.all_reduce_population_count(m)

    pltpu.sync_copy(sv, surv_hbm.at[pl.ds(wid * SR, SR)])
    stv[...] = offv
    pltpu.sync_copy(stv, cnt_hbm.at[pl.ds(wid * 16, 16)])

    @pl.when(wid == 0)
    def _():
        stv[...] = jnp.where(lane == 0, b0, jnp.where(lane == 1, kp, zeros))
        pltpu.sync_copy(stv, st_hbm)


@functools.partial(
    pl.kernel,
    mesh=_mesh,
    compiler_params=_params,
    out_type=jax.ShapeDtypeStruct((16,), jnp.float32),
    scratch_types=[
        pltpu.VMEM((NW * SR,), jnp.int32),
        pltpu.VMEM((HCHUNK,), jnp.float32),
        pltpu.VMEM((16 * NB2,), jnp.int32),
        pltpu.VMEM((NB2,), jnp.int32),
        pltpu.VMEM((NW * 16,), jnp.int32),
        pltpu.VMEM((16,), jnp.int32),
        pltpu.VMEM((16,), jnp.float32),
        pltpu.SemaphoreType.DMA,
    ],
)
def _kc(err_hbm, surv_hbm, cnt_hbm, st_hbm, res_hbm,
        sb, fb, hb, hrow, cntv, stv, outv, sem):
    wid = _wid()

    @pl.when(wid == 0)
    def _():
        pltpu.sync_copy(cnt_hbm, cntv)
        pltpu.sync_copy(st_hbm, stv)
        svec = stv[...]
        b0 = svec[0]
        kp = svec[1]
        lane = _lane()
        zeros = jnp.zeros((16,), jnp.int32)
        ones = jnp.ones((16,), jnp.int32)

        cps = pltpu.async_copy(surv_hbm, sb, sem)
        mx = zeros
        for r in range(NW):
            mx = jnp.maximum(mx, cntv[pl.ds(r * 16, 16)])
        ov = jnp.any(mx > CAP)
        cps.wait()

        def survivors_hist(match_fn):
            for r in range(NW):
                cnt = cntv[pl.ds(r * 16, 16)][0]

                @plsc.parallel_loop(0, (cnt + 15) >> 4)
                def _(j):
                    u = sb[pl.ds(r * SR + j * 16, 16)]
                    valid = (j * 16 + lane) < cnt
                    m, bkt = match_fn(u)
                    plsc.addupdate_scatter(
                        hb, [lane * NB2 + bkt], ones, mask=m & valid)

        def full_rescan_hist(match_fn):
            def rbody(r, _):
                for c in range(2):
                    pltpu.sync_copy(
                        err_hbm.at[pl.ds(r * C + c * HCHUNK, HCHUNK)], fb)

                    @functools.partial(
                        lax.fori_loop, 0, HCHUNK // 16, init_val=0)
                    def _(j, _ignored):
                        e = fb[pl.ds(j * 16, 16)]
                        u = lax.bitcast_convert_type(e, jnp.int32)
                        m, bkt = match_fn(u)
                        plsc.addupdate_scatter(
                            hb, [lane * NB2 + bkt], ones, mask=m)
                        return 0
                return 0

            lax.fori_loop(0, NW, rbody, 0)

        def select_pass(match_fn, kth):
            _zero_ref(hb, 16 * NB2)
            pl.when(~ov)(lambda: survivors_hist(match_fn))
            pl.when(ov)(lambda: full_rescan_hist(match_fn))
            _reduce_banked(hb, hrow, NB2)
            return _scan_rows(hrow, 1, NB2, kth)

        b1, pre1 = select_pass(
            lambda u: (((u >> 20) == b0), (u >> 10) & (NB2 - 1)), kp)
        p01 = b0 * NB2 + b1
        kpp = kp - pre1
        b2, _unused = select_pass(
            lambda u: (((u >> 10) == p01), u & (NB3 - 1)), kpp)
        bits = p01 * NB3 + b2
        outv[...] = lax.bitcast_convert_type(zeros + bits, jnp.float32)
        pltpu.sync_copy(outv, res_hbm)


def kernel(pred, target):
    padp = jnp.concatenate(
        [pred, jnp.full((NPAD - N,), jnp.inf, jnp.float32)])
    padt = jnp.concatenate([target, jnp.zeros((NPAD - N,), jnp.float32)])
    err, h1 = _k1(padp, padt)
    surv, cnt, st = _k2(err, h1)
    res = _kc(err, surv, cnt, st)
    return res[0]


# speedup vs baseline: 1.1335x; 1.0267x over previous
import functools

import jax
import jax.numpy as jnp
from jax import lax
from jax.experimental import pallas as pl
from jax.experimental.pallas import tpu as pltpu
from jax.experimental.pallas import tpu_sc as plsc

N = 1_000_000
K = 950_000
NW = 32
C = 31_264
NPAD = NW * C
NB1 = 2048
NB2 = 1024
NB3 = 1024
NS = 16
C2 = 2 * C
SR = 4096
CAP = SR - 16

_mesh = plsc.VectorSubcoreMesh(core_axis_name="c", subcore_axis_name="s")
_mesh1 = plsc.VectorSubcoreMesh(
    core_axis_name="c", subcore_axis_name="s", num_cores=1)
_params = pltpu.CompilerParams(needs_layout_passes=False)


def _lane():
    return lax.broadcasted_iota(jnp.int32, (16,), 0)


def _zero_ref(ref, nwords):
    z = jnp.zeros((16,), jnp.int32)

    @plsc.parallel_loop(0, nwords // 16, unroll=8)
    def _(i):
        ref[pl.ds(i * 16, 16)] = z


def _scan_rows(hin, nrows, nbins, kth):
    zeros = jnp.zeros((16,), jnp.int32)
    ones = jnp.ones((16,), jnp.int32)

    @plsc.parallel_loop(
        0, nbins // 16, carry=(jnp.int32(0), jnp.int32(0), jnp.int32(0)))
    def body(g, carry):
        total, b_acc, pre_acc = carry
        acc = zeros
        for r in range(nrows):
            acc = acc + hin[pl.ds(r * nbins + g * 16, 16)]
        cs = plsc.cumsum(acc)
        lt = (total + cs) < kth
        b_acc = b_acc + jnp.sum(jnp.where(lt, ones, zeros))
        pre_acc = pre_acc + jnp.sum(jnp.where(lt, acc, zeros))
        total = total + jnp.sum(acc)
        return total, b_acc, pre_acc

    _, b, pre = body
    return b, pre


def _reduce_banked(hb, hrow, nbins):
    zeros = jnp.zeros((16,), jnp.int32)

    @plsc.parallel_loop(0, nbins // 16, unroll=2)
    def _(g):
        acc = zeros
        for l in range(16):
            acc = acc + hb[pl.ds(l * nbins + g * 16, 16)]
        hrow[pl.ds(g * 16, 16)] = acc


@functools.partial(
    pl.kernel,
    mesh=_mesh,
    compiler_params=_params,
    out_type=(
        jax.ShapeDtypeStruct((NPAD,), jnp.float32),
        jax.ShapeDtypeStruct((NW * NB1,), jnp.int32),
    ),
    scratch_types=[
        pltpu.VMEM((C,), jnp.float32),
        pltpu.VMEM((C,), jnp.float32),
        pltpu.VMEM((16 * NB1,), jnp.int32),
        pltpu.VMEM((NB1,), jnp.int32),
        pltpu.SemaphoreType.DMA,
        pltpu.SemaphoreType.DMA,
    ],
)
def _k1(p_hbm, t_hbm, err_hbm, h1_hbm, pv, tv, hb, hrow, sem1, sem2):
    wid = lax.axis_index("s") * 2 + lax.axis_index("c")
    base = wid * C
    cp1 = pltpu.async_copy(p_hbm.at[pl.ds(base, C)], pv, sem1)
    cp2 = pltpu.async_copy(t_hbm.at[pl.ds(base, C)], tv, sem2)
    _zero_ref(hb, 16 * NB1)
    cp1.wait()
    cp2.wait()
    lane_off = _lane() * NB1
    ones = jnp.ones((16,), jnp.int32)

    @plsc.parallel_loop(0, C // 16, unroll=8)
    def _(i):
        p = pv[pl.ds(i * 16, 16)]
        t = tv[pl.ds(i * 16, 16)]
        e = jnp.abs(p - t)
        pv[pl.ds(i * 16, 16)] = e
        u = lax.bitcast_convert_type(e, jnp.int32)
        off = lane_off + (u >> 20)
        plsc.addupdate_scatter(hb, [off], ones)

    _reduce_banked(hb, hrow, NB1)
    pltpu.sync_copy(pv, err_hbm.at[pl.ds(base, C)])
    pltpu.sync_copy(hrow, h1_hbm.at[pl.ds(wid * NB1, NB1)])


@functools.partial(
    pl.kernel,
    mesh=_mesh1,
    compiler_params=_params,
    out_type=jax.ShapeDtypeStruct((16,), jnp.float32),
    scratch_types=[
        pltpu.VMEM((C,), jnp.float32),
        pltpu.VMEM((NW * NB1,), jnp.int32),
        pltpu.VMEM((SR,), jnp.int32),
        pltpu.VMEM((SR,), jnp.int32),
        pltpu.VMEM((16 * NB2,), jnp.int32),
        pltpu.VMEM((NB2,), jnp.int32),
        pltpu.VMEM((NS * 16,), jnp.int32),
        pltpu.VMEM((16,), jnp.int32),
        pltpu.VMEM((16,), jnp.float32),
        pltpu.VMEM_SHARED((NS * SR,), jnp.int32),
        pltpu.VMEM_SHARED((NS * 16,), jnp.int32),
        pltpu.SemaphoreType.DMA,
        pltpu.SemaphoreType.DMA,
    ],
)
def _k2(err_hbm, h1_hbm, res_hbm, ev, hin, sv, buf, hb, hrow, cbuf, stv,
        outv, sh_s, sh_c, sem1, sem2):
    sid = lax.axis_index("s")
    base = sid * C2
    lane = _lane()
    zeros = jnp.zeros((16,), jnp.int32)
    ones = jnp.ones((16,), jnp.int32)
    capv = zeros + CAP

    cp1 = pltpu.async_copy(err_hbm.at[pl.ds(base, C)], ev, sem1)
    cp2 = pltpu.async_copy(h1_hbm, hin, sem2)
    cp2.wait()
    b0, pre0 = _scan_rows(hin, NW, NB1, K)
    kp = K - pre0

    def compact_half(offv):
        @plsc.parallel_loop(0, C // 16, carry=offv)
        def offv2(i, offv):
            e = ev[pl.ds(i * 16, 16)]
            u = lax.bitcast_convert_type(e, jnp.int32)
            m = (u >> 20) == b0
            mi = jnp.where(m, ones, zeros)
            pos = offv + plsc.cumsum(mi) - mi
            plsc.store_scatter(sv, [jnp.minimum(pos, capv)], u, mask=m)
            return offv + plsc.all_reduce_population_count(m)

        return offv2

    cp1.wait()
    offv = compact_half(zeros)
    pltpu.sync_copy(err_hbm.at[pl.ds(base + C, C)], ev)
    offv = compact_half(offv)

    pltpu.sync_copy(sv, sh_s.at[pl.ds(sid * SR, SR)])
    stv[...] = offv
    pltpu.sync_copy(stv, sh_c.at[pl.ds(sid * 16, 16)])
    plsc.subcore_barrier()

    @pl.when(sid == 0)
    def _():
        pltpu.sync_copy(sh_c, cbuf)
        mx = zeros
        for r in range(NS):
            mx = jnp.maximum(mx, cbuf[pl.ds(r * 16, 16)])
        ov = jnp.any(mx > CAP)

        def survivors_hist(match_fn):
            for r in range(NS):
                cnt = cbuf[pl.ds(r * 16, 16)][0]
                pltpu.sync_copy(sh_s.at[pl.ds(r * SR, SR)], buf)

                @plsc.parallel_loop(0, (cnt + 15) >> 4)
                def _(j):
                    u = buf[pl.ds(j * 16, 16)]
                    valid = (j * 16 + lane) < cnt
                    m, bkt = match_fn(u)
                    plsc.addupdate_scatter(
                        hb, [lane * NB2 + bkt], ones, mask=m & valid)

        def full_rescan_hist(match_fn):
            def rbody(r, _):
                pltpu.sync_copy(err_hbm.at[pl.ds(r * C, C)], ev)

                @functools.partial(lax.fori_loop, 0, C // 16, init_val=0)
                def _(j, _ignored):
                    e = ev[pl.ds(j * 16, 16)]
                    u = lax.bitcast_convert_type(e, jnp.int32)
                    m, bkt = match_fn(u)
                    plsc.addupdate_scatter(
                        hb, [lane * NB2 + bkt], ones, mask=m)
                    return 0
                return 0

            lax.fori_loop(0, NW, rbody, 0)

        def select_pass(match_fn, kth):
            _zero_ref(hb, 16 * NB2)
            pl.when(~ov)(lambda: survivors_hist(match_fn))
            pl.when(ov)(lambda: full_rescan_hist(match_fn))
            _reduce_banked(hb, hrow, NB2)
            return _scan_rows(hrow, 1, NB2, kth)

        b1, pre1 = select_pass(
            lambda u: (((u >> 20) == b0), (u >> 10) & (NB2 - 1)), kp)
        p01 = b0 * NB2 + b1
        kpp = kp - pre1
        b2, _unused = select_pass(
            lambda u: (((u >> 10) == p01), u & (NB3 - 1)), kpp)
        bits = p01 * NB3 + b2
        outv[...] = lax.bitcast_convert_type(zeros + bits, jnp.float32)
        pltpu.sync_copy(outv, res_hbm)


def kernel(pred, target):
    padp = jnp.concatenate(
        [pred, jnp.full((NPAD - N,), jnp.inf, jnp.float32)])
    padt = jnp.concatenate([target, jnp.zeros((NPAD - N,), jnp.float32)])
    err, h1 = _k1(padp, padt)
    res = _k2(err, h1)
    return res[0]


# speedup vs baseline: 1.1784x; 1.0396x over previous
import functools

import jax
import jax.numpy as jnp
from jax import lax
from jax.experimental import pallas as pl
from jax.experimental.pallas import tpu as pltpu
from jax.experimental.pallas import tpu_sc as plsc

N = 1_000_000
K = 950_000
NW = 32
C = 31_264
NPAD = NW * C
NB1 = 2048
NB2 = 1024
NB3 = 1024
SR = 2048
CAP = SR - 16
HCHUNK = C // 2

_mesh = plsc.VectorSubcoreMesh(core_axis_name="c", subcore_axis_name="s")
_params = pltpu.CompilerParams(needs_layout_passes=False)


def _wid():
    return lax.axis_index("s") * 2 + lax.axis_index("c")


def _lane():
    return lax.broadcasted_iota(jnp.int32, (16,), 0)


def _zero_ref(ref, nwords):
    z = jnp.zeros((16,), jnp.int32)

    @plsc.parallel_loop(0, nwords // 16, unroll=8)
    def _(i):
        ref[pl.ds(i * 16, 16)] = z


def _scan_rows(hin, nrows, nbins, kth):
    zeros = jnp.zeros((16,), jnp.int32)
    ones = jnp.ones((16,), jnp.int32)

    @plsc.parallel_loop(
        0, nbins // 16, carry=(jnp.int32(0), jnp.int32(0), jnp.int32(0)))
    def body(g, carry):
        total, b_acc, pre_acc = carry
        acc = zeros
        for r in range(nrows):
            acc = acc + hin[pl.ds(r * nbins + g * 16, 16)]
        cs = plsc.cumsum(acc)
        lt = (total + cs) < kth
        b_acc = b_acc + jnp.sum(jnp.where(lt, ones, zeros))
        pre_acc = pre_acc + jnp.sum(jnp.where(lt, acc, zeros))
        total = total + jnp.sum(acc)
        return total, b_acc, pre_acc

    _, b, pre = body
    return b, pre


def _reduce_banked(hb, hrow, nbins):
    zeros = jnp.zeros((16,), jnp.int32)

    @plsc.parallel_loop(0, nbins // 16, unroll=2)
    def _(g):
        acc = zeros
        for l in range(16):
            acc = acc + hb[pl.ds(l * nbins + g * 16, 16)]
        hrow[pl.ds(g * 16, 16)] = acc


@functools.partial(
    pl.kernel,
    mesh=_mesh,
    compiler_params=_params,
    out_type=(
        jax.ShapeDtypeStruct((NPAD,), jnp.float32),
        jax.ShapeDtypeStruct((NW * NB1,), jnp.int32),
    ),
    scratch_types=[
        pltpu.VMEM((C,), jnp.float32),
        pltpu.VMEM((C,), jnp.float32),
        pltpu.VMEM((16 * NB1,), jnp.int32),
        pltpu.VMEM((NB1,), jnp.int32),
        pltpu.SemaphoreType.DMA,
        pltpu.SemaphoreType.DMA,
    ],
)
def _k1(p_hbm, t_hbm, err_hbm, h1_hbm, pv, tv, hb, hrow, sem1, sem2):
    wid = _wid()
    base = wid * C
    cp1 = pltpu.async_copy(p_hbm.at[pl.ds(base, C - 448)],
                           pv.at[pl.ds(0, C - 448)], sem1)
    cp2 = pltpu.async_copy(t_hbm.at[pl.ds(base, C - 448)],
                           tv.at[pl.ds(0, C - 448)], sem2)
    _zero_ref(hb, 16 * NB1)
    cp1.wait()
    cp2.wait()

    @pl.when(wid != NW - 1)
    def _():
        pltpu.sync_copy(p_hbm.at[pl.ds(base + C - 448, 448)],
                        pv.at[pl.ds(C - 448, 448)])
        pltpu.sync_copy(t_hbm.at[pl.ds(base + C - 448, 448)],
                        tv.at[pl.ds(C - 448, 448)])

    @pl.when(wid == NW - 1)
    def _():
        inf = jnp.full((16,), jnp.inf, jnp.float32)
        zf = jnp.zeros((16,), jnp.float32)
        for j in range(448 // 16):
            pv[pl.ds(C - 448 + j * 16, 16)] = inf
            tv[pl.ds(C - 448 + j * 16, 16)] = zf
    lane_off = _lane() * NB1
    ones = jnp.ones((16,), jnp.int32)

    @plsc.parallel_loop(0, C // 16, unroll=8)
    def _(i):
        p = pv[pl.ds(i * 16, 16)]
        t = tv[pl.ds(i * 16, 16)]
        e = jnp.abs(p - t)
        pv[pl.ds(i * 16, 16)] = e
        u = lax.bitcast_convert_type(e, jnp.int32)
        off = lane_off + (u >> 20)
        plsc.addupdate_scatter(hb, [off], ones)

    _reduce_banked(hb, hrow, NB1)
    pltpu.sync_copy(pv, err_hbm.at[pl.ds(base, C)])
    pltpu.sync_copy(hrow, h1_hbm.at[pl.ds(wid * NB1, NB1)])


@functools.partial(
    pl.kernel,
    mesh=_mesh,
    compiler_params=_params,
    out_type=(
        jax.ShapeDtypeStruct((NW * SR,), jnp.int32),
        jax.ShapeDtypeStruct((NW * 16,), jnp.int32),
        jax.ShapeDtypeStruct((16,), jnp.int32),
    ),
    scratch_types=[
        pltpu.VMEM((C,), jnp.float32),
        pltpu.VMEM((NW * NB1,), jnp.int32),
        pltpu.VMEM((SR,), jnp.int32),
        pltpu.VMEM((16,), jnp.int32),
        pltpu.SemaphoreType.DMA,
        pltpu.SemaphoreType.DMA,
    ],
)
def _k2(err_hbm, h1_hbm, surv_hbm, cnt_hbm, st_hbm, ev, hin, sv, stv,
        sem1, sem2):
    wid = _wid()
    base = wid * C
    cp1 = pltpu.async_copy(err_hbm.at[pl.ds(base, C)], ev, sem1)
    cp2 = pltpu.async_copy(h1_hbm, hin, sem2)
    cp2.wait()
    b0, pre0 = _scan_rows(hin, NW, NB1, K)
    kp = K - pre0
    cp1.wait()
    lane = _lane()
    zeros = jnp.zeros((16,), jnp.int32)
    ones = jnp.ones((16,), jnp.int32)
    capv = zeros + CAP

    @plsc.parallel_loop(0, C // 16, carry=zeros)
    def offv(i, offv):
        e = ev[pl.ds(i * 16, 16)]
        u = lax.bitcast_convert_type(e, jnp.int32)
        m = (u >> 20) == b0
        mi = jnp.where(m, ones, zeros)
        pos = offv + plsc.cumsum(mi) - mi
        plsc.store_scatter(sv, [jnp.minimum(pos, capv)], u, mask=m)
        return offv + plsc.all_reduce_population_count(m)

    pltpu.sync_copy(sv, surv_hbm.at[pl.ds(wid * SR, SR)])
    stv[...] = offv
    pltpu.sync_copy(stv, cnt_hbm.at[pl.ds(wid * 16, 16)])

    @pl.when(wid == 0)
    def _():
        stv[...] = jnp.where(lane == 0, b0, jnp.where(lane == 1, kp, zeros))
        pltpu.sync_copy(stv, st_hbm)


@functools.partial(
    pl.kernel,
    mesh=_mesh,
    compiler_params=_params,
    out_type=jax.ShapeDtypeStruct((16,), jnp.float32),
    scratch_types=[
        pltpu.VMEM((NW * SR,), jnp.int32),
        pltpu.VMEM((HCHUNK,), jnp.float32),
        pltpu.VMEM((16 * NB2,), jnp.int32),
        pltpu.VMEM((NB2,), jnp.int32),
        pltpu.VMEM((NW * 16,), jnp.int32),
        pltpu.VMEM((16,), jnp.int32),
        pltpu.VMEM((16,), jnp.float32),
        pltpu.SemaphoreType.DMA,
    ],
)
def _kc(err_hbm, surv_hbm, cnt_hbm, st_hbm, res_hbm,
        sb, fb, hb, hrow, cntv, stv, outv, sem):
    wid = _wid()

    @pl.when(wid == 0)
    def _():
        pltpu.sync_copy(cnt_hbm, cntv)
        pltpu.sync_copy(st_hbm, stv)
        svec = stv[...]
        b0 = svec[0]
        kp = svec[1]
        lane = _lane()
        zeros = jnp.zeros((16,), jnp.int32)
        ones = jnp.ones((16,), jnp.int32)

        cps = pltpu.async_copy(surv_hbm, sb, sem)
        mx = zeros
        for r in range(NW):
            mx = jnp.maximum(mx, cntv[pl.ds(r * 16, 16)])
        ov = jnp.any(mx > CAP)
        cps.wait()

        def survivors_hist(match_fn):
            for r in range(NW):
                cnt = cntv[pl.ds(r * 16, 16)][0]

                @plsc.parallel_loop(0, (cnt + 15) >> 4)
                def _(j):
                    u = sb[pl.ds(r * SR + j * 16, 16)]
                    valid = (j * 16 + lane) < cnt
                    m, bkt = match_fn(u)
                    plsc.addupdate_scatter(
                        hb, [lane * NB2 + bkt], ones, mask=m & valid)

        def full_rescan_hist(match_fn):
            def rbody(r, _):
                for c in range(2):
                    pltpu.sync_copy(
                        err_hbm.at[pl.ds(r * C + c * HCHUNK, HCHUNK)], fb)

                    @functools.partial(
                        lax.fori_loop, 0, HCHUNK // 16, init_val=0)
                    def _(j, _ignored):
                        e = fb[pl.ds(j * 16, 16)]
                        u = lax.bitcast_convert_type(e, jnp.int32)
                        m, bkt = match_fn(u)
                        plsc.addupdate_scatter(
                            hb, [lane * NB2 + bkt], ones, mask=m)
                        return 0
                return 0

            lax.fori_loop(0, NW, rbody, 0)

        def select_pass(match_fn, kth):
            _zero_ref(hb, 16 * NB2)
            pl.when(~ov)(lambda: survivors_hist(match_fn))
            pl.when(ov)(lambda: full_rescan_hist(match_fn))
            _reduce_banked(hb, hrow, NB2)
            return _scan_rows(hrow, 1, NB2, kth)

        b1, pre1 = select_pass(
            lambda u: (((u >> 20) == b0), (u >> 10) & (NB2 - 1)), kp)
        p01 = b0 * NB2 + b1
        kpp = kp - pre1
        b2, _unused = select_pass(
            lambda u: (((u >> 10) == p01), u & (NB3 - 1)), kpp)
        bits = p01 * NB3 + b2
        outv[...] = lax.bitcast_convert_type(zeros + bits, jnp.float32)
        pltpu.sync_copy(outv, res_hbm)


def kernel(pred, target):
    err, h1 = _k1(pred, target)
    surv, cnt, st = _k2(err, h1)
    res = _kc(err, surv, cnt, st)
    return res[0]


# speedup vs baseline: 1.2600x; 1.0692x over previous
import functools

import jax
import jax.numpy as jnp
from jax import lax
from jax.experimental import pallas as pl
from jax.experimental.pallas import tpu as pltpu
from jax.experimental.pallas import tpu_sc as plsc

N = 1_000_000
K = 950_000
NW = 32
C = 31_264
NPAD = NW * C
NB1 = 2048
NB2 = 1024
NB3 = 1024
SR = 1792
CAP = SR - 16
HCHUNK = C // 2

_mesh = plsc.VectorSubcoreMesh(core_axis_name="c", subcore_axis_name="s")
_params = pltpu.CompilerParams(needs_layout_passes=False)


def _wid():
    return lax.axis_index("s") * 2 + lax.axis_index("c")


def _lane():
    return lax.broadcasted_iota(jnp.int32, (16,), 0)


def _zero_ref(ref, nwords):
    z = jnp.zeros((16,), jnp.int32)

    @plsc.parallel_loop(0, nwords // 16, unroll=8)
    def _(i):
        ref[pl.ds(i * 16, 16)] = z


def _scan_rows(hin, nrows, nbins, kth):
    zeros = jnp.zeros((16,), jnp.int32)
    ones = jnp.ones((16,), jnp.int32)

    @plsc.parallel_loop(
        0, nbins // 16, carry=(jnp.int32(0), jnp.int32(0), jnp.int32(0)))
    def body(g, carry):
        total, b_acc, pre_acc = carry
        acc = zeros
        for r in range(nrows):
            acc = acc + hin[pl.ds(r * nbins + g * 16, 16)]
        cs = plsc.cumsum(acc)
        lt = (total + cs) < kth
        b_acc = b_acc + jnp.sum(jnp.where(lt, ones, zeros))
        pre_acc = pre_acc + jnp.sum(jnp.where(lt, acc, zeros))
        total = total + jnp.sum(acc)
        return total, b_acc, pre_acc

    _, b, pre = body
    return b, pre


def _reduce_banked(hb, hrow, nbins):
    zeros = jnp.zeros((16,), jnp.int32)

    @plsc.parallel_loop(0, nbins // 16, unroll=2)
    def _(g):
        acc = zeros
        for l in range(16):
            acc = acc + hb[pl.ds(l * nbins + g * 16, 16)]
        hrow[pl.ds(g * 16, 16)] = acc


@functools.partial(
    pl.kernel,
    mesh=_mesh,
    compiler_params=_params,
    out_type=(
        jax.ShapeDtypeStruct((NPAD,), jnp.float32),
        jax.ShapeDtypeStruct((NW * NB1,), jnp.int32),
    ),
    scratch_types=[
        pltpu.VMEM((C,), jnp.float32),
        pltpu.VMEM((C,), jnp.float32),
        pltpu.VMEM((16 * NB1,), jnp.int32),
        pltpu.VMEM((NB1,), jnp.int32),
        pltpu.SemaphoreType.DMA,
        pltpu.SemaphoreType.DMA,
    ],
)
def _k1(p_hbm, t_hbm, err_hbm, h1_hbm, pv, tv, hb, hrow, sem1, sem2):
    wid = _wid()
    base = wid * C
    cp1 = pltpu.async_copy(p_hbm.at[pl.ds(base, C - 448)],
                           pv.at[pl.ds(0, C - 448)], sem1)
    cp2 = pltpu.async_copy(t_hbm.at[pl.ds(base, C - 448)],
                           tv.at[pl.ds(0, C - 448)], sem2)
    _zero_ref(hb, 16 * NB1)
    cp1.wait()
    cp2.wait()

    @pl.when(wid != NW - 1)
    def _():
        pltpu.sync_copy(p_hbm.at[pl.ds(base + C - 448, 448)],
                        pv.at[pl.ds(C - 448, 448)])
        pltpu.sync_copy(t_hbm.at[pl.ds(base + C - 448, 448)],
                        tv.at[pl.ds(C - 448, 448)])

    @pl.when(wid == NW - 1)
    def _():
        inf = jnp.full((16,), jnp.inf, jnp.float32)
        zf = jnp.zeros((16,), jnp.float32)
        for j in range(448 // 16):
            pv[pl.ds(C - 448 + j * 16, 16)] = inf
            tv[pl.ds(C - 448 + j * 16, 16)] = zf
    lane_off = _lane() * NB1
    ones = jnp.ones((16,), jnp.int32)

    @plsc.parallel_loop(0, C // 16, unroll=8)
    def _(i):
        p = pv[pl.ds(i * 16, 16)]
        t = tv[pl.ds(i * 16, 16)]
        e = jnp.abs(p - t)
        pv[pl.ds(i * 16, 16)] = e
        u = lax.bitcast_convert_type(e, jnp.int32)
        off = lane_off + (u >> 20)
        plsc.addupdate_scatter(hb, [off], ones)

    _reduce_banked(hb, hrow, NB1)
    pltpu.sync_copy(pv, err_hbm.at[pl.ds(base, C)])
    pltpu.sync_copy(hrow, h1_hbm.at[pl.ds(wid * NB1, NB1)])


@functools.partial(
    pl.kernel,
    mesh=_mesh,
    compiler_params=_params,
    out_type=(
        jax.ShapeDtypeStruct((NW * SR,), jnp.int32),
        jax.ShapeDtypeStruct((NW * 16,), jnp.int32),
        jax.ShapeDtypeStruct((16,), jnp.int32),
    ),
    scratch_types=[
        pltpu.VMEM((C,), jnp.float32),
        pltpu.VMEM((NW * NB1,), jnp.int32),
        pltpu.VMEM((SR,), jnp.int32),
        pltpu.VMEM((16,), jnp.int32),
        pltpu.SemaphoreType.DMA,
        pltpu.SemaphoreType.DMA,
    ],
)
def _k2(err_hbm, h1_hbm, surv_hbm, cnt_hbm, st_hbm, ev, hin, sv, stv,
        sem1, sem2):
    wid = _wid()
    base = wid * C
    cp1 = pltpu.async_copy(err_hbm.at[pl.ds(base, C)], ev, sem1)
    cp2 = pltpu.async_copy(h1_hbm, hin, sem2)
    cp2.wait()
    b0, pre0 = _scan_rows(hin, NW, NB1, K)
    kp = K - pre0
    cp1.wait()
    lane = _lane()
    zeros = jnp.zeros((16,), jnp.int32)

    @plsc.parallel_loop(0, C // 16, carry=jnp.int32(0))
    def off(i, off):
        e = ev[pl.ds(i * 16, 16)]
        u = lax.bitcast_convert_type(e, jnp.int32)
        m = (u >> 20) == b0
        plsc.store_compressed(
            sv.at[pl.ds(jnp.minimum(off, CAP), 16)], u, mask=m)
        return off + plsc.all_reduce_population_count(m)[0]

    pltpu.sync_copy(sv, surv_hbm.at[pl.ds(wid * SR, SR)])
    stv[...] = zeros + off
    pltpu.sync_copy(stv, cnt_hbm.at[pl.ds(wid * 16, 16)])

    @pl.when(wid == 0)
    def _():
        stv[...] = jnp.where(lane == 0, b0, jnp.where(lane == 1, kp, zeros))
        pltpu.sync_copy(stv, st_hbm)


@functools.partial(
    pl.kernel,
    mesh=_mesh,
    compiler_params=_params,
    out_type=jax.ShapeDtypeStruct((NW * NB2,), jnp.int32),
    scratch_types=[
        pltpu.VMEM((SR,), jnp.int32),
        pltpu.VMEM((16 * NB2,), jnp.int32),
        pltpu.VMEM((NB2,), jnp.int32),
        pltpu.VMEM((16,), jnp.int32),
        pltpu.SemaphoreType.DMA,
    ],
)
def _k3(surv_hbm, cnt_hbm, h2_hbm, sv, hb, hrow, stv, sem):
    wid = _wid()
    cp = pltpu.async_copy(surv_hbm.at[pl.ds(wid * SR, SR)], sv, sem)
    pltpu.sync_copy(cnt_hbm.at[pl.ds(wid * 16, 16)], stv)
    _zero_ref(hb, 16 * NB2)
    cnt = jnp.minimum(stv[...][0], CAP)
    lane = _lane()
    ones = jnp.ones((16,), jnp.int32)
    cp.wait()

    @plsc.parallel_loop(0, (cnt + 15) >> 4)
    def _(j):
        u = sv[pl.ds(j * 16, 16)]
        valid = (j * 16 + lane) < cnt
        bkt = (u >> 10) & (NB2 - 1)
        plsc.addupdate_scatter(hb, [lane * NB2 + bkt], ones, mask=valid)

    _reduce_banked(hb, hrow, NB2)
    pltpu.sync_copy(hrow, h2_hbm.at[pl.ds(wid * NB2, NB2)])


@functools.partial(
    pl.kernel,
    mesh=_mesh,
    compiler_params=_params,
    out_type=jax.ShapeDtypeStruct((16,), jnp.float32),
    scratch_types=[
        pltpu.VMEM((NW * SR,), jnp.int32),
        pltpu.VMEM((NW * NB2,), jnp.int32),
        pltpu.VMEM((HCHUNK,), jnp.float32),
        pltpu.VMEM((16 * NB3,), jnp.int32),
        pltpu.VMEM((NB3,), jnp.int32),
        pltpu.VMEM((NW * 16,), jnp.int32),
        pltpu.VMEM((16,), jnp.int32),
        pltpu.VMEM((16,), jnp.float32),
        pltpu.SemaphoreType.DMA,
        pltpu.SemaphoreType.DMA,
    ],
)
def _k4(err_hbm, surv_hbm, cnt_hbm, st_hbm, h2_hbm, res_hbm,
        sb, h2v, fb, hb, hrow, cntv, stv, outv, sem1, sem2):
    wid = _wid()

    @pl.when(wid == 0)
    def _():
        cps = pltpu.async_copy(surv_hbm, sb, sem1)
        cph = pltpu.async_copy(h2_hbm, h2v, sem2)
        pltpu.sync_copy(cnt_hbm, cntv)
        pltpu.sync_copy(st_hbm, stv)
        svec = stv[...]
        b0 = svec[0]
        kp = svec[1]
        lane = _lane()
        zeros = jnp.zeros((16,), jnp.int32)
        ones = jnp.ones((16,), jnp.int32)
        mx = zeros
        for r in range(NW):
            mx = jnp.maximum(mx, cntv[pl.ds(r * 16, 16)])
        ov = jnp.any(mx > CAP)
        cph.wait()

        def full_rescan_hist(match_fn):
            def rbody(c, _):
                pltpu.sync_copy(err_hbm.at[pl.ds(c * HCHUNK, HCHUNK)], fb)

                @functools.partial(
                    lax.fori_loop, 0, HCHUNK // 16, init_val=0)
                def _(j, _ignored):
                    e = fb[pl.ds(j * 16, 16)]
                    u = lax.bitcast_convert_type(e, jnp.int32)
                    m, bkt = match_fn(u)
                    plsc.addupdate_scatter(
                        hb, [lane * NB3 + bkt], ones, mask=m)
                    return 0
                return 0

            lax.fori_loop(0, NPAD // HCHUNK, rbody, 0)

        match2 = lambda u: (((u >> 20) == b0), (u >> 10) & (NB2 - 1))

        def scan_h2():
            return _scan_rows(h2v, NW, NB2, kp)

        def rescan_h2():
            _zero_ref(hb, 16 * NB3)
            full_rescan_hist(match2)
            _reduce_banked(hb, hrow, NB3)
            return _scan_rows(hrow, 1, NB2, kp)

        b1, pre1 = lax.cond(ov, rescan_h2, scan_h2)
        p01 = b0 * NB2 + b1
        kpp = kp - pre1

        match3 = lambda u: (((u >> 10) == p01), u & (NB3 - 1))
        _zero_ref(hb, 16 * NB3)
        cps.wait()

        @pl.when(~ov)
        def _():
            for r in range(NW):
                cnt = cntv[pl.ds(r * 16, 16)][0]

                @plsc.parallel_loop(0, (cnt + 15) >> 4)
                def _(j):
                    u = sb[pl.ds(r * SR + j * 16, 16)]
                    valid = (j * 16 + lane) < cnt
                    m, bkt = match3(u)
                    plsc.addupdate_scatter(
                        hb, [lane * NB3 + bkt], ones, mask=m & valid)

        pl.when(ov)(lambda: full_rescan_hist(match3))
        _reduce_banked(hb, hrow, NB3)
        b2, _unused = _scan_rows(hrow, 1, NB3, kpp)
        bits = p01 * NB3 + b2
        outv[...] = lax.bitcast_convert_type(zeros + bits, jnp.float32)
        pltpu.sync_copy(outv, res_hbm)


def kernel(pred, target):
    err, h1 = _k1(pred, target)
    surv, cnt, st = _k2(err, h1)
    h2 = _k3(surv, cnt)
    res = _k4(err, surv, cnt, st, h2)
    return res[0]
